# baseline (speedup 1.0000x reference)
import functools
import math

import jax
import jax.numpy as jnp
from jax import lax
from jax.experimental import pallas as pl
from jax.experimental.pallas import tpu as pltpu
from jax.experimental.pallas import tpu_sc as plsc

D_MODEL = 64
SCALE = math.sqrt(D_MODEL)
LANES = 16
C = 128
NBUF = 4


@functools.cache
def _build(n_rows):
    info = plsc.get_sparse_core_info()
    NC, NS = info.num_cores, info.num_subcores
    NW = NC * NS
    assert n_rows % NW == 0
    n_chunks = n_rows // NW
    assert n_chunks % NBUF == 0

    mesh = plsc.VectorSubcoreMesh(core_axis_name="c", subcore_axis_name="s")

    @functools.partial(
        pl.kernel,
        mesh=mesh,
        out_type=jax.ShapeDtypeStruct((n_rows // 32, 8, 32, C * 8), jnp.float32),
        scratch_types=[
            pltpu.VMEM((n_chunks, C), jnp.int32),
        ]
        + [pltpu.VMEM((C, D_MODEL), jnp.float32) for _ in range(NBUF)]
        + [pltpu.VMEM((8, C * 8), jnp.float32) for _ in range(NBUF)]
        + [pltpu.SemaphoreType.DMA for _ in range(2 * NBUF)],
        compiler_params=pltpu.CompilerParams(
            use_tc_tiling_on_sc=False, needs_layout_passes=False
        ),
    )
    def emb_kernel(x2_hbm, lut_hbm, out_hbm, idx_v, *bufs_and_sems):
        in_bufs = bufs_and_sems[0:NBUF]
        out_bufs = bufs_and_sems[NBUF : 2 * NBUF]
        gsems = bufs_and_sems[2 * NBUF : 3 * NBUF]
        ssems = bufs_and_sems[3 * NBUF : 4 * NBUF]

        wid = lax.axis_index("s") * NC + lax.axis_index("c")
        cbase = wid * n_chunks

        pltpu.sync_copy(x2_hbm.at[pl.ds(cbase, n_chunks)], idx_v)

        b_iota = lax.iota(jnp.int32, LANES)
        rowv = []
        colv = []
        for j in range(D_MODEL // LANES):
            dvec = b_iota + (j * LANES)
            rowv.append(dvec // 8)
            colv.append((dvec % 8) * C)

        def gather(b, g):
            return pltpu.make_async_copy(
                lut_hbm.at[idx_v.at[g]], in_bufs[b], gsems[b]
            )

        def scatter(b, g):
            r = cbase + g
            t = (r // 256) * 8 + (r % 8)
            bhi = (r // 8) % 32
            return pltpu.make_async_copy(
                out_bufs[b], out_hbm.at[t, :, bhi], ssems[b]
            )

        for b in range(NBUF):
            gather(b, b).start()

        @pl.loop(0, n_chunks, step=NBUF)
        def _(g0):
            for b in range(NBUF):
                g = g0 + b
                gather(b, g).wait()

                @pl.when(g >= NBUF)
                def _():
                    scatter(b, g - NBUF).wait()

                src = in_bufs[b]
                dst = out_bufs[b]

                @plsc.parallel_loop(0, C, unroll=8)
                def _(bb):
                    for j in range(D_MODEL // LANES):
                        vals = src[bb, pl.ds(j * LANES, LANES)] * SCALE
                        plsc.store_scatter(dst, [rowv[j], colv[j] + bb], vals)

                scatter(b, g).start()

                @pl.when(g + NBUF < n_chunks)
                def _():
                    gather(b, g + NBUF).start()

        for b in range(NBUF):
            scatter(b, n_chunks - NBUF + b).wait()

    return emb_kernel


def kernel(x, lut):
    B, T = x.shape
    xt = jnp.transpose(x).astype(jnp.int32)
    x4 = jnp.transpose(jnp.reshape(xt, (T // 8, 8, B // C, C)), (0, 2, 1, 3))
    x2 = jnp.reshape(x4, (-1, C))
    raw = _build(x2.shape[0])(x2, lut)
    o5 = jnp.reshape(raw, (T, 8, B // C, 8, C))
    out = jnp.transpose(o5, (2, 4, 0, 1, 3))
    return jnp.reshape(out, (B, T, D_MODEL))


# speedup vs baseline: 1.1665x; 1.1665x over previous
import functools
import math

import jax
import jax.numpy as jnp
from jax import lax
from jax.experimental import pallas as pl
from jax.experimental.pallas import tpu as pltpu
from jax.experimental.pallas import tpu_sc as plsc

D_MODEL = 64
SCALE = math.sqrt(D_MODEL)
LANES = 16
C = 128
NBUF = 4
PITCH = 65


@functools.cache
def _build(n_rows):
    info = plsc.get_sparse_core_info()
    NC, NS = info.num_cores, info.num_subcores
    NW = NC * NS
    assert n_rows % NW == 0
    n_chunks = n_rows // NW
    assert n_chunks % NBUF == 0

    mesh = plsc.VectorSubcoreMesh(core_axis_name="c", subcore_axis_name="s")

    @functools.partial(
        pl.kernel,
        mesh=mesh,
        out_type=jax.ShapeDtypeStruct((n_rows // 32, 8, 32, C * 8), jnp.float32),
        scratch_types=[
            pltpu.VMEM((n_chunks, C), jnp.int32),
            pltpu.VMEM(((C // LANES) * LANES * PITCH,), jnp.float32),
        ]
        + [pltpu.VMEM((C, D_MODEL), jnp.float32) for _ in range(NBUF)]
        + [pltpu.VMEM((8, C * 8), jnp.float32) for _ in range(NBUF)]
        + [pltpu.SemaphoreType.DMA for _ in range(2 * NBUF)],
        compiler_params=pltpu.CompilerParams(
            use_tc_tiling_on_sc=False, needs_layout_passes=False
        ),
    )
    def emb_kernel(x2_hbm, lut_hbm, out_hbm, idx_v, stage, *bufs_and_sems):
        in_bufs = bufs_and_sems[0:NBUF]
        out_bufs = bufs_and_sems[NBUF : 2 * NBUF]
        gsems = bufs_and_sems[2 * NBUF : 3 * NBUF]
        ssems = bufs_and_sems[3 * NBUF : 4 * NBUF]

        wid = lax.axis_index("s") * NC + lax.axis_index("c")
        cbase = wid * n_chunks

        pltpu.sync_copy(x2_hbm.at[pl.ds(cbase, n_chunks)], idx_v)

        col_iota = lax.iota(jnp.int32, LANES) * PITCH

        def start_gather(b, g):
            pltpu.make_async_copy(
                lut_hbm.at[idx_v.at[g]], in_bufs[b], gsems[b]
            ).start()

        def wait_gather(b, g):
            pltpu.make_async_copy(
                lut_hbm.at[idx_v.at[g]], in_bufs[b], gsems[b]
            ).wait()

        def scatter(b, g):
            r = cbase + g
            t = (r // 256) * 8 + (r % 8)
            bhi = (r // 8) % 32
            return pltpu.make_async_copy(
                out_bufs[b], out_hbm.at[t, :, bhi], ssems[b]
            )

        for b in range(NBUF):
            start_gather(b, b)

        @pl.loop(0, n_chunks, step=NBUF)
        def _(g0):
            for b in range(NBUF):
                g = g0 + b
                wait_gather(b, g)

                @pl.when(g >= NBUF)
                def _():
                    scatter(b, g - NBUF).wait()

                src = in_bufs[b]
                dst = out_bufs[b]

                @plsc.parallel_loop(0, C // LANES, unroll=1)
                def _(bg):
                    b0 = bg * LANES
                    sbase = bg * (LANES * PITCH)
                    for bb in range(LANES):
                        for j in range(D_MODEL // LANES):
                            s = pl.ds(j * LANES, LANES)
                            stage[pl.ds(sbase + bb * PITCH + j * LANES, LANES)] = (
                                src[b0 + bb, s] * SCALE
                            )
                    for d in range(D_MODEL):
                        vals = plsc.load_gather(stage, [col_iota + (sbase + d)])
                        dst[d // 8, pl.ds((d % 8) * C + b0, LANES)] = vals

                scatter(b, g).start()

                @pl.when(g + NBUF < n_chunks)
                def _():
                    start_gather(b, g + NBUF)

        for b in range(NBUF):
            scatter(b, n_chunks - NBUF + b).wait()

    return emb_kernel


def kernel(x, lut):
    B, T = x.shape
    V = lut.shape[0]
    xt = jnp.transpose(x).astype(jnp.int32)
    x4 = jnp.transpose(jnp.reshape(xt, (T // 8, 8, B // C, C)), (0, 2, 1, 3))
    x2 = jnp.reshape(x4, (-1, C))
    del V
    raw = _build(x2.shape[0])(x2, lut)
    o5 = jnp.reshape(raw, (T, 8, B // C, 8, C))
    out = jnp.transpose(o5, (2, 4, 0, 1, 3))
    return jnp.reshape(out, (B, T, D_MODEL))


# speedup vs baseline: 1.2155x; 1.0420x over previous
import functools
import math

import jax
import jax.numpy as jnp
from jax import lax
from jax.experimental import pallas as pl
from jax.experimental.pallas import tpu as pltpu
from jax.experimental.pallas import tpu_sc as plsc

D_MODEL = 64
SCALE = math.sqrt(D_MODEL)
LANES = 16
C = 128
NBUF = 5
PITCH = 65


@functools.cache
def _build(n_rows):
    info = plsc.get_sparse_core_info()
    NC, NS = info.num_cores, info.num_subcores
    NW = NC * NS
    assert n_rows % NW == 0
    n_chunks = n_rows // NW
    assert n_chunks % NBUF == 0

    mesh = plsc.VectorSubcoreMesh(core_axis_name="c", subcore_axis_name="s")

    @functools.partial(
        pl.kernel,
        mesh=mesh,
        out_type=jax.ShapeDtypeStruct((n_rows // 32, 8, 32, C * 8), jnp.float32),
        scratch_types=[
            pltpu.VMEM((n_chunks, C), jnp.int32),
            pltpu.VMEM(((C // LANES) * LANES * PITCH,), jnp.float32),
        ]
        + [pltpu.VMEM((C, D_MODEL), jnp.float32) for _ in range(NBUF)]
        + [pltpu.VMEM((8, C * 8), jnp.float32) for _ in range(NBUF)]
        + [pltpu.SemaphoreType.DMA for _ in range(2 * NBUF)],
        compiler_params=pltpu.CompilerParams(
            use_tc_tiling_on_sc=False, needs_layout_passes=False
        ),
    )
    def emb_kernel(x2_hbm, lut_hbm, out_hbm, idx_v, stage, *bufs_and_sems):
        in_bufs = bufs_and_sems[0:NBUF]
        out_bufs = bufs_and_sems[NBUF : 2 * NBUF]
        gsems = bufs_and_sems[2 * NBUF : 3 * NBUF]
        ssems = bufs_and_sems[3 * NBUF : 4 * NBUF]

        wid = lax.axis_index("s") * NC + lax.axis_index("c")
        cbase = wid * n_chunks

        pltpu.sync_copy(x2_hbm.at[pl.ds(cbase, n_chunks)], idx_v)

        col_iota = lax.iota(jnp.int32, LANES) * PITCH

        def start_gather(b, g):
            pltpu.make_async_copy(
                lut_hbm.at[idx_v.at[g]], in_bufs[b], gsems[b]
            ).start()

        def wait_gather(b, g):
            pltpu.make_async_copy(
                lut_hbm.at[idx_v.at[g]], in_bufs[b], gsems[b]
            ).wait()

        def scatter(b, g):
            r = cbase + g
            t = (r // 256) * 8 + (r % 8)
            bhi = (r // 8) % 32
            return pltpu.make_async_copy(
                out_bufs[b], out_hbm.at[t, :, bhi], ssems[b]
            )

        for b in range(NBUF):
            start_gather(b, b)

        @pl.loop(0, n_chunks, step=NBUF)
        def _(g0):
            for b in range(NBUF):
                g = g0 + b
                wait_gather(b, g)

                @pl.when(g >= NBUF)
                def _():
                    scatter(b, g - NBUF).wait()

                src = in_bufs[b]
                dst = out_bufs[b]

                @plsc.parallel_loop(0, C // LANES, unroll=2)
                def _(bg):
                    b0 = bg * LANES
                    sbase = bg * (LANES * PITCH)
                    for bb in range(LANES):
                        for j in range(D_MODEL // LANES):
                            s = pl.ds(j * LANES, LANES)
                            stage[pl.ds(sbase + bb * PITCH + j * LANES, LANES)] = (
                                src[b0 + bb, s] * SCALE
                            )
                    for d in range(D_MODEL):
                        vals = plsc.load_gather(stage, [col_iota + (sbase + d)])
                        dst[d // 8, pl.ds((d % 8) * C + b0, LANES)] = vals

                scatter(b, g).start()

                @pl.when(g + NBUF < n_chunks)
                def _():
                    start_gather(b, g + NBUF)

        for b in range(NBUF):
            scatter(b, n_chunks - NBUF + b).wait()

    return emb_kernel


def kernel(x, lut):
    B, T = x.shape
    V = lut.shape[0]
    xt = jnp.transpose(x).astype(jnp.int32)
    x4 = jnp.transpose(jnp.reshape(xt, (T // 8, 8, B // C, C)), (0, 2, 1, 3))
    x2 = jnp.reshape(x4, (-1, C))
    del V
    raw = _build(x2.shape[0])(x2, lut)
    o5 = jnp.reshape(raw, (T, 8, B // C, 8, C))
    out = jnp.transpose(o5, (2, 4, 0, 1, 3))
    return jnp.reshape(out, (B, T, D_MODEL))
